# baseline (speedup 1.0000x reference)
import functools
import math

import jax
import jax.numpy as jnp
from jax import lax
from jax.experimental import pallas as pl
from jax.experimental.pallas import tpu as pltpu
from jax.experimental.pallas import tpu_sc as plsc

VOCAB = 256
EMB = 128
HID = 768
B = 16
L = 2048

NC = 2
NS = 16
NW = NC * NS
LSPAN = L // NW
CHUNK = 32
NLANE = 16


def _proj_body(t_ref, w_ref, b_ref, o_ref):
    o_ref[...] = (
        jnp.dot(t_ref[...], w_ref[...], preferred_element_type=jnp.float32)
        + b_ref[...]
    )


def _project(table, W, b):
    return pl.pallas_call(
        _proj_body,
        out_shape=jax.ShapeDtypeStruct((VOCAB, HID), jnp.float32),
    )(table, W, b.reshape(1, HID))


def _pos_encoding():
    position = jnp.arange(0, L, dtype=jnp.float32)[:, None]
    div_term = jnp.exp(
        jnp.arange(0, HID, 2, dtype=jnp.float32) * (-math.log(10000.0) / HID)
    )
    pe = jnp.zeros((L, HID), dtype=jnp.float32)
    pe = pe.at[:, 0::2].set(jnp.sin(position * div_term))
    pe = pe.at[:, 1::2].set(jnp.cos(position * div_term))
    return pe


_sc_mesh = plsc.VectorSubcoreMesh(core_axis_name="c", subcore_axis_name="s")


@functools.partial(
    pl.kernel,
    mesh=_sc_mesh,
    out_type=jax.ShapeDtypeStruct((B, L, HID), jnp.float32),
    scratch_types=[
        pltpu.VMEM((LSPAN, HID), jnp.float32),
        pltpu.VMEM((B * LSPAN,), jnp.int32),
        pltpu.VMEM((CHUNK, HID), jnp.float32),
    ],
)
def _lookup(p_hbm, pe_hbm, idx_hbm, out_hbm, pe_v, idx_v, buf_v):
    wid = lax.axis_index("s") * NC + lax.axis_index("c")
    l0 = wid * LSPAN

    pltpu.sync_copy(pe_hbm.at[pl.ds(l0, LSPAN)], pe_v)
    pltpu.sync_copy(idx_hbm.at[pl.ds(wid * (B * LSPAN), B * LSPAN)], idx_v)

    @pl.loop(0, B)
    def _batch(bi):
        @pl.loop(0, LSPAN // CHUNK)
        def _chunk(j):
            pltpu.sync_copy(
                p_hbm.at[idx_v.at[pl.ds(bi * LSPAN + j * CHUNK, CHUNK)]], buf_v
            )

            @pl.loop(0, CHUNK)
            def _row(r):
                lr = j * CHUNK + r
                for c in range(0, HID, NLANE):
                    buf_v[r, pl.ds(c, NLANE)] = (
                        buf_v[r, pl.ds(c, NLANE)] + pe_v[lr, pl.ds(c, NLANE)]
                    )

            pltpu.sync_copy(
                buf_v, out_hbm.at[bi, pl.ds(l0 + j * CHUNK, CHUNK)]
            )


def kernel(phonemes, table, W, b):
    P = _project(table, W, b)
    pe = _pos_encoding()
    idx = (
        phonemes.astype(jnp.int32)
        .reshape(B, NW, LSPAN)
        .transpose(1, 0, 2)
        .reshape(NW * B * LSPAN)
    )
    return _lookup(P, pe, idx)


# speedup vs baseline: 2.0531x; 2.0531x over previous
import functools
import math

import jax
import jax.numpy as jnp
from jax import lax
from jax.experimental import pallas as pl
from jax.experimental.pallas import tpu as pltpu
from jax.experimental.pallas import tpu_sc as plsc

VOCAB = 256
EMB = 128
HID = 768
B = 16
L = 2048

NC = 2
NS = 16
NW = NC * NS
LSPAN = L // NW
CHUNK = 16
CPB = LSPAN // CHUNK
NCHUNK = B * CPB
NBUF = 4
DIST = 2
NLANE = 16


def _proj_body(t_ref, w_ref, b_ref, o_ref):
    o_ref[...] = (
        jnp.dot(t_ref[...], w_ref[...], preferred_element_type=jnp.float32)
        + b_ref[...]
    )


def _project(table, W, b):
    return pl.pallas_call(
        _proj_body,
        out_shape=jax.ShapeDtypeStruct((VOCAB, HID), jnp.float32),
    )(table, W, b.reshape(1, HID))


def _pos_encoding():
    position = jnp.arange(0, L, dtype=jnp.float32)[:, None]
    div_term = jnp.exp(
        jnp.arange(0, HID, 2, dtype=jnp.float32) * (-math.log(10000.0) / HID)
    )
    pe = jnp.zeros((L, HID), dtype=jnp.float32)
    pe = pe.at[:, 0::2].set(jnp.sin(position * div_term))
    pe = pe.at[:, 1::2].set(jnp.cos(position * div_term))
    return pe


_sc_mesh = plsc.VectorSubcoreMesh(core_axis_name="c", subcore_axis_name="s")


@functools.partial(
    pl.kernel,
    mesh=_sc_mesh,
    out_type=jax.ShapeDtypeStruct((B, L, HID), jnp.float32),
    scratch_types=[
        pltpu.VMEM((LSPAN, HID), jnp.float32),
        pltpu.VMEM((B * LSPAN,), jnp.int32),
        pltpu.VMEM((NBUF, CHUNK, HID), jnp.float32),
        pltpu.SemaphoreType.DMA((NBUF,)),
        pltpu.SemaphoreType.DMA((NBUF,)),
    ],
)
def _lookup(p_hbm, pe_hbm, idx_hbm, out_hbm, pe_v, idx_v, buf_v, gsem, ssem):
    wid = lax.axis_index("s") * NC + lax.axis_index("c")
    l0 = wid * LSPAN

    def issue_gather(c, k):
        pltpu.async_copy(
            p_hbm.at[idx_v.at[pl.ds(c * CHUNK, CHUNK)]], buf_v.at[k], gsem.at[k]
        )

    def wait_gather(k):
        pltpu.make_async_copy(
            p_hbm.at[pl.ds(0, CHUNK)], buf_v.at[k], gsem.at[k]
        ).wait()

    def issue_store(bi, k):
        pltpu.async_copy(
            buf_v.at[k], out_hbm.at[bi, pl.ds(l0 + k * CHUNK, CHUNK)], ssem.at[k]
        )

    def wait_store(k):
        pltpu.make_async_copy(
            buf_v.at[k], out_hbm.at[0, pl.ds(l0, CHUNK)], ssem.at[k]
        ).wait()

    pltpu.sync_copy(idx_hbm.at[pl.ds(wid * (B * LSPAN), B * LSPAN)], idx_v)
    for c0 in range(DIST):
        issue_gather(c0, c0)
    pltpu.sync_copy(pe_hbm.at[pl.ds(l0, LSPAN)], pe_v)

    @pl.loop(0, NCHUNK, step=NBUF)
    def _slot(t):
        bi = t // CPB
        for k in range(NBUF):
            tt = t + k
            nx = tt + DIST
            kn = (k + DIST) % NBUF

            @pl.when(nx < NCHUNK)
            def _ahead():
                @pl.when(nx >= NBUF)
                def _drain():
                    wait_store(kn)

                issue_gather(nx, kn)

            wait_gather(k)

            @pl.loop(0, CHUNK)
            def _row(r):
                for c in range(0, HID, NLANE):
                    plsc.addupdate(
                        buf_v.at[k, r, pl.ds(c, NLANE)],
                        pe_v[k * CHUNK + r, pl.ds(c, NLANE)],
                    )

            issue_store(bi, k)

    for k in range(NBUF):
        wait_store(k)


def kernel(phonemes, table, W, b):
    P = _project(table, W, b)
    pe = _pos_encoding()
    idx = (
        phonemes.astype(jnp.int32)
        .reshape(B, NW, LSPAN)
        .transpose(1, 0, 2)
        .reshape(NW * B * LSPAN)
    )
    return _lookup(P, pe, idx)
